# Initial kernel scaffold; baseline (speedup 1.0000x reference)
import functools

import jax
import jax.numpy as jnp
from jax import lax
from jax.experimental import pallas as pl
from jax.experimental.pallas import tpu as pltpu
from jax.experimental.pallas import tpu_sc as plsc

NC = 2
NS = 16
L = 16
NW = NC * NS

N = 10000
E = 160000
D = 256
B = 1024
TEMP = 0.07

EPT = -(-E // NW)
EPT_PAD = -(-EPT // L) * L
EPAD = NW * EPT_PAD
NGROUPS = EPT_PAD // L

ROWS = 2 * B + 256
RPT = ROWS // NS
CNT_FLAT = ROWS * 16
CPT = CNT_FLAT // NS
JUNK = 2 * B * 16
K = 64
LIST_LEN = EPT_PAD + 112
PPT = B // NW


def _sc_body(nf, dst, nbr, rel, bids, neg1, zrows, zflat,
             out_sums, out_cnt, out_slot, out_self,
             slot_tab, bids_v, dst_v, nbr_v, rel_v, nbr_c, pkd_c,
             rows_a, sf_rows, nbr_b, tgt_b, cidx_b, ones_k,
             slotp_v, posi_v, accum_sp, cnt_sp, sem1, sem2):
    sid = lax.axis_index("s")
    cid = lax.axis_index("c")
    wid = sid * NC + cid

    pltpu.sync_copy(zrows, accum_sp.at[pl.ds(sid * RPT, RPT)])
    pltpu.sync_copy(zflat, cnt_sp.at[pl.ds(sid * CPT, CPT)])
    pltpu.sync_copy(neg1, slot_tab)
    pltpu.sync_copy(bids, bids_v)

    def build(g, carry):
        bv = bids_v[pl.ds(g * L, L)]
        vals = g * L + lax.iota(jnp.int32, (L,))
        plsc.store_scatter(slot_tab, [bv], vals)
        return carry

    lax.fori_loop(0, B // L, build, 0)

    def prefill(g, carry):
        nbr_c[pl.ds(g * L, L)] = jnp.zeros((L,), jnp.int32)
        pkd_c[pl.ds(g * L, L)] = jnp.full((L,), JUNK, jnp.int32)
        return carry

    lax.fori_loop(0, LIST_LEN // L, prefill, 0)
    for t in range(K // L):
        ones_k[pl.ds(t * L, L)] = jnp.ones((L,), jnp.float32)

    plsc.subcore_barrier()

    ebase = wid * EPT_PAD
    pltpu.sync_copy(dst.at[pl.ds(ebase, EPT_PAD)], dst_v)
    pltpu.sync_copy(nbr.at[pl.ds(ebase, EPT_PAD)], nbr_v)
    pltpu.sync_copy(rel.at[pl.ds(ebase, EPT_PAD)], rel_v)
    lane = lax.iota(jnp.int32, (L,))

    def scan(j, off):
        base = j * L
        dstv = dst_v[pl.ds(base, L)]
        relv = rel_v[pl.ds(base, L)]
        valid = (ebase + base + lane) < E
        dcl = jnp.minimum(jnp.maximum(dstv, 0), N - 1)
        slots = plsc.load_gather(slot_tab, [dcl])
        hit = valid & (slots >= 0)
        v1m = (relv == 0) | (relv >= 7)
        tgt = slots + jnp.where(v1m, 0, B)
        pkd = tgt * 16 + relv
        nbrv = nbr_v[pl.ds(base, L)]
        plsc.store_compressed(nbr_c.at[pl.ds(off, L)], nbrv, mask=hit)
        plsc.store_compressed(pkd_c.at[pl.ds(off, L)], pkd, mask=hit)
        nhit = plsc.all_reduce_population_count(hit)
        return off + nhit[0]

    off = lax.fori_loop(0, NGROUPS, scan, jnp.int32(0))
    nb = (off + (K - 1)) // K

    def process(j, carry):
        base = j * K
        for t in range(K // L):
            nbr_b[pl.ds(t * L, L)] = nbr_c[pl.ds(base + t * L, L)]
            pk = pkd_c[pl.ds(base + t * L, L)]
            tgt_b[pl.ds(t * L, L)] = pk // 16
            cidx_b[pl.ds(t * L, L)] = pk
        pltpu.async_copy(nf.at[nbr_b], rows_a, sem1).wait()
        pltpu.sync_copy(rows_a, accum_sp.at[tgt_b], add=True)
        pltpu.sync_copy(ones_k, cnt_sp.at[cidx_b], add=True)
        return carry

    lax.fori_loop(0, nb, process, 0)

    pbase = wid * PPT
    for t in range(PPT // L):
        bv = bids_v[pl.ds(pbase + t * L, L)]
        slotp_v[pl.ds(t * L, L)] = plsc.load_gather(slot_tab, [bv])
        posi_v[pl.ds(t * L, L)] = bv
    pltpu.sync_copy(slotp_v, out_slot.at[pl.ds(pbase, PPT)])
    pltpu.async_copy(nf.at[posi_v], sf_rows, sem2).wait()
    pltpu.sync_copy(sf_rows, out_self.at[pl.ds(pbase, PPT)])

    plsc.subcore_barrier()

    pltpu.sync_copy(accum_sp.at[pl.ds(sid * RPT, RPT)],
                    out_sums.at[pl.ds(cid * ROWS + sid * RPT, RPT)])
    pltpu.sync_copy(cnt_sp.at[pl.ds(sid * CPT, CPT)],
                    out_cnt.at[pl.ds(cid * CNT_FLAT + sid * CPT, CPT)])


_sc_aggregate = functools.partial(
    pl.kernel,
    out_type=(
        jax.ShapeDtypeStruct((NC * ROWS, D), jnp.float32),
        jax.ShapeDtypeStruct((NC * CNT_FLAT,), jnp.float32),
        jax.ShapeDtypeStruct((B,), jnp.int32),
        jax.ShapeDtypeStruct((B, D), jnp.float32),
    ),
    mesh=plsc.VectorSubcoreMesh(core_axis_name="c", subcore_axis_name="s",
                                num_cores=NC, num_subcores=NS),
    scratch_types=(
        pltpu.VMEM((N,), jnp.int32),
        pltpu.VMEM((B,), jnp.int32),
        pltpu.VMEM((EPT_PAD,), jnp.int32),
        pltpu.VMEM((EPT_PAD,), jnp.int32),
        pltpu.VMEM((EPT_PAD,), jnp.int32),
        pltpu.VMEM((LIST_LEN,), jnp.int32),
        pltpu.VMEM((LIST_LEN,), jnp.int32),
        pltpu.VMEM((K, D), jnp.float32),
        pltpu.VMEM((PPT, D), jnp.float32),
        pltpu.VMEM((K,), jnp.int32),
        pltpu.VMEM((K,), jnp.int32),
        pltpu.VMEM((K,), jnp.int32),
        pltpu.VMEM((K,), jnp.float32),
        pltpu.VMEM((PPT,), jnp.int32),
        pltpu.VMEM((PPT,), jnp.int32),
        pltpu.VMEM_SHARED((ROWS, D), jnp.float32),
        pltpu.VMEM_SHARED((CNT_FLAT,), jnp.float32),
        pltpu.SemaphoreType.DMA,
        pltpu.SemaphoreType.DMA,
    ),
)(_sc_body)


def _tc_head(sums_ref, cnt_ref, slotp_ref, self_ref, rel16_ref,
             w1a_ref, b1a_ref, w1b_ref, b1b_ref,
             w2a_ref, b2a_ref, w2b_ref, b2b_ref, out_ref):
    f32 = jnp.float32
    hi = lax.Precision.HIGHEST

    s = sums_ref[0] + sums_ref[1]
    c = cnt_ref[0] + cnt_ref[1]
    slotp = slotp_ref[...]
    selff = self_ref[...]
    rel16 = rel16_ref[...]

    pcols = lax.broadcasted_iota(jnp.int32, (B, B), 1)
    P = (slotp == pcols).astype(f32)

    def view(s_rows, c_rows):
        sp = jnp.dot(P, s_rows, precision=hi, preferred_element_type=f32)
        cp = jnp.dot(P, c_rows, precision=hi, preferred_element_type=f32)
        r = jnp.dot(cp, rel16, precision=hi, preferred_element_type=f32)
        n = jnp.sum(cp, axis=1, keepdims=True)
        return jnp.where(n > 0.0, (sp + r) / jnp.maximum(n, 1.0), selff)

    v1 = view(s[0:B], c[0:B])
    v2 = view(s[B:2 * B], c[B:2 * B])

    def proj(v, wa, ba, wb, bb):
        h = jnp.maximum(jnp.dot(v, wa, precision=hi,
                                preferred_element_type=f32) + ba, 0.0)
        return jnp.dot(h, wb, precision=hi, preferred_element_type=f32) + bb

    z1 = proj(v1, w1a_ref[...], b1a_ref[...], w1b_ref[...], b1b_ref[...])
    z2 = proj(v2, w2a_ref[...], b2a_ref[...], w2b_ref[...], b2b_ref[...])

    def normalize(z):
        nrm = jnp.sqrt(jnp.sum(z * z, axis=1, keepdims=True))
        return z / jnp.maximum(nrm, 1e-12)

    z1n = normalize(z1)
    z2n = normalize(z2)
    sim = lax.dot_general(z1n, z2n, (((1,), (1,)), ((), ())),
                          precision=hi, preferred_element_type=f32) / TEMP

    mx_r = jnp.max(sim, axis=1, keepdims=True)
    lse_r = mx_r[:, 0] + jnp.log(jnp.sum(jnp.exp(sim - mx_r), axis=1))
    mx_c = jnp.max(sim, axis=0, keepdims=True)
    lse_c = mx_c[0, :] + jnp.log(jnp.sum(jnp.exp(sim - mx_c), axis=0))
    eye = (lax.broadcasted_iota(jnp.int32, (B, B), 0) == pcols).astype(f32)
    diag_mean = jnp.sum(sim * eye) / B
    loss = 0.5 * (jnp.mean(lse_r) + jnp.mean(lse_c)) - diag_mean
    out_ref[0, 0] = loss


def kernel(node_features, edge_index, edge_rel, batch_node_ids, rel_emb,
           W1a, b1a, W1b, b1b, W2a, b2a, W2b, b2b):
    nf = node_features.astype(jnp.float32)
    dst = jnp.pad(edge_index[0].astype(jnp.int32), (0, EPAD - E))
    nbr = jnp.pad(edge_index[1].astype(jnp.int32), (0, EPAD - E))
    rel = jnp.pad(edge_rel.astype(jnp.int32), (0, EPAD - E))
    bids = batch_node_ids.astype(jnp.int32)
    neg1 = jnp.full((N,), -1, jnp.int32)
    zrows = jnp.zeros((RPT, D), jnp.float32)
    zflat = jnp.zeros((CPT,), jnp.float32)

    sums, cnt, slotp, selff = _sc_aggregate(nf, dst, nbr, rel, bids,
                                            neg1, zrows, zflat)

    rel16 = jnp.zeros((16, D), jnp.float32).at[:rel_emb.shape[0]].set(rel_emb)
    loss = pl.pallas_call(
        _tc_head,
        out_shape=jax.ShapeDtypeStruct((1, 1), jnp.float32),
    )(sums.reshape(NC, ROWS, D), cnt.reshape(NC, ROWS, 16),
      slotp.reshape(B, 1), selff, rel16,
      W1a, b1a.reshape(1, D), W1b, b1b.reshape(1, D // 2),
      W2a, b2a.reshape(1, D), W2b, b2b.reshape(1, D // 2))
    return loss[0, 0]


# speedup vs baseline: 6.7039x; 6.7039x over previous
import functools

import jax
import jax.numpy as jnp
from jax import lax
from jax.experimental import pallas as pl
from jax.experimental.pallas import tpu as pltpu
from jax.experimental.pallas import tpu_sc as plsc

NC = 2
NS = 16
L = 16
NW = NC * NS

N = 10000
E = 160000
D = 256
B = 1024
TEMP = 0.07

EPT = -(-E // NW)
EPT_PAD = -(-EPT // L) * L
EPAD = NW * EPT_PAD
NGROUPS = EPT_PAD // L

ROWS = 2 * B + 256
RPT = ROWS // NS
JUNK = 2 * B * 16
K = 64
LIST_LEN = EPT_PAD + 112
PPT = B // NW


def _sc_body(nf, dst, nbr, rel, bids, neg1, zrows, z128,
             out_sums, out_cnt, out_slot, out_self,
             slot_tab, bids_v, dst_v, nbr_v, rel_v, nbr_c, pkd_c,
             rows_a, oh_buf, sf_rows, nbr_b, tgt_b, ohidx_b,
             slotp_v, posi_v, sem1, sem2):
    sid = lax.axis_index("s")
    cid = lax.axis_index("c")
    wid = sid * NC + cid
    lane = lax.iota(jnp.int32, L)

    pltpu.sync_copy(zrows, out_sums.at[pl.ds(cid * ROWS + sid * RPT, RPT)])
    pltpu.sync_copy(z128, out_cnt.at[pl.ds(cid * ROWS + sid * RPT, RPT)])
    pltpu.sync_copy(neg1, slot_tab)
    pltpu.sync_copy(bids, bids_v)

    def build(g, carry):
        bv = bids_v[pl.ds(g * L, L)]
        vals = g * L + lane
        plsc.store_scatter(slot_tab, [bv], vals)
        return carry

    lax.fori_loop(0, B // L, build, 0)

    def prefill(g, carry):
        nbr_c[pl.ds(g * L, L)] = jnp.zeros((L,), jnp.int32)
        pkd_c[pl.ds(g * L, L)] = jnp.full((L,), JUNK, jnp.int32)
        return carry

    lax.fori_loop(0, LIST_LEN // L, prefill, 0)

    def zero_oh(g, carry):
        for col in range(256 // L):
            oh_buf[g, pl.ds(col * L, L)] = jnp.zeros((L,), jnp.float32)
        return carry

    lax.fori_loop(0, K, zero_oh, 0)

    plsc.subcore_barrier()

    ebase = wid * EPT_PAD
    pltpu.sync_copy(dst.at[pl.ds(ebase, EPT_PAD)], dst_v)
    pltpu.sync_copy(nbr.at[pl.ds(ebase, EPT_PAD)], nbr_v)
    pltpu.sync_copy(rel.at[pl.ds(ebase, EPT_PAD)], rel_v)

    def scan(j, off):
        base = j * L
        dstv = dst_v[pl.ds(base, L)]
        relv = rel_v[pl.ds(base, L)]
        valid = (ebase + base + lane) < E
        dcl = jnp.minimum(jnp.maximum(dstv, 0), N - 1)
        slots = plsc.load_gather(slot_tab, [dcl])
        hit = valid & (slots >= 0)
        v1m = (relv == 0) | (relv >= 7)
        tgt = slots + jnp.where(v1m, 0, B)
        pkd = tgt * 16 + relv
        nbrv = nbr_v[pl.ds(base, L)]
        plsc.store_compressed(nbr_c.at[pl.ds(off, L)], nbrv, mask=hit)
        plsc.store_compressed(pkd_c.at[pl.ds(off, L)], pkd, mask=hit)
        nhit = plsc.all_reduce_population_count(hit)
        return off + nhit[0]

    off = lax.fori_loop(0, NGROUPS, scan, jnp.int32(0))
    nb = (off + (K - 1)) // K

    def process(j, carry):
        base = j * K
        for t in range(K // L):
            nbr_b[pl.ds(t * L, L)] = nbr_c[pl.ds(base + t * L, L)]
            pk = pkd_c[pl.ds(base + t * L, L)]
            tgt_b[pl.ds(t * L, L)] = pk // 16 + cid * ROWS
            ohidx_b[pl.ds(t * L, L)] = pk & 15
        pltpu.async_copy(nf.at[nbr_b], rows_a, sem1).wait()
        for t in range(K // L):
            oi = ohidx_b[pl.ds(t * L, L)]
            plsc.store_scatter(oh_buf, [t * L + lane, oi],
                               jnp.ones((L,), jnp.float32))
        copies = []
        for t in range(K // L):
            tv = tgt_b[pl.ds(t * L, L)]
            copies.append(pltpu.async_copy(
                rows_a.at[pl.ds(t * L, L)], out_sums.at[tv], sem1, add=True))
            copies.append(pltpu.async_copy(
                oh_buf.at[pl.ds(t * L, L)], out_cnt.at[tv], sem2, add=True))
        for cp in copies:
            cp.wait()
        for t in range(K // L):
            oi = ohidx_b[pl.ds(t * L, L)]
            plsc.store_scatter(oh_buf, [t * L + lane, oi],
                               jnp.zeros((L,), jnp.float32))
        return carry

    lax.fori_loop(0, nb, process, 0)

    pbase = wid * PPT
    for t in range(PPT // L):
        bv = bids_v[pl.ds(pbase + t * L, L)]
        slotp_v[pl.ds(t * L, L)] = plsc.load_gather(slot_tab, [bv])
        posi_v[pl.ds(t * L, L)] = bv
    pltpu.sync_copy(slotp_v, out_slot.at[pl.ds(pbase, PPT)])
    pltpu.async_copy(nf.at[posi_v], sf_rows, sem2).wait()
    pltpu.sync_copy(sf_rows, out_self.at[pl.ds(pbase, PPT)])


_sc_aggregate = None


def _make_sc_aggregate():
    global _sc_aggregate
    if _sc_aggregate is None:
        _sc_aggregate = pl.kernel(
            _sc_body,
            out_type=(
                jax.ShapeDtypeStruct((NC * ROWS, D), jnp.float32),
                jax.ShapeDtypeStruct((NC * ROWS, 256), jnp.float32),
                jax.ShapeDtypeStruct((B,), jnp.int32),
                jax.ShapeDtypeStruct((B, D), jnp.float32),
            ),
            mesh=plsc.VectorSubcoreMesh(core_axis_name="c",
                                        subcore_axis_name="s",
                                        num_cores=NC, num_subcores=NS),
            scratch_types=(
                pltpu.VMEM((N,), jnp.int32),
                pltpu.VMEM((B,), jnp.int32),
                pltpu.VMEM((EPT_PAD,), jnp.int32),
                pltpu.VMEM((EPT_PAD,), jnp.int32),
                pltpu.VMEM((EPT_PAD,), jnp.int32),
                pltpu.VMEM((LIST_LEN,), jnp.int32),
                pltpu.VMEM((LIST_LEN,), jnp.int32),
                pltpu.VMEM((K, D), jnp.float32),
                pltpu.VMEM((K, 256), jnp.float32),
                pltpu.VMEM((PPT, D), jnp.float32),
                pltpu.VMEM((K,), jnp.int32),
                pltpu.VMEM((K,), jnp.int32),
                pltpu.VMEM((K,), jnp.int32),
                pltpu.VMEM((PPT,), jnp.int32),
                pltpu.VMEM((PPT,), jnp.int32),
                pltpu.SemaphoreType.DMA,
                pltpu.SemaphoreType.DMA,
            ),
            compiler_params=pltpu.CompilerParams(needs_layout_passes=False),
        )
    return _sc_aggregate


def _tc_head(sums_ref, cnt_ref, slotp_ref, self_ref, rel16_ref,
             w1a_ref, b1a_ref, w1b_ref, b1b_ref,
             w2a_ref, b2a_ref, w2b_ref, b2b_ref, out_ref):
    f32 = jnp.float32
    hi = lax.Precision.HIGHEST

    s = sums_ref[0] + sums_ref[1]
    c = cnt_ref[0] + cnt_ref[1]
    slotp = slotp_ref[...]
    selff = self_ref[...]
    rel16 = rel16_ref[...]

    pcols = lax.broadcasted_iota(jnp.int32, (B, B), 1)
    P = (slotp == pcols).astype(f32)

    def view(s_rows, c_rows):
        sp = jnp.dot(P, s_rows, precision=hi, preferred_element_type=f32)
        cp = jnp.dot(P, c_rows, precision=hi, preferred_element_type=f32)
        r = jnp.dot(cp[:, 0:16], rel16, precision=hi,
                    preferred_element_type=f32)
        n = jnp.sum(cp, axis=1, keepdims=True)
        return jnp.where(n > 0.0, (sp + r) / jnp.maximum(n, 1.0), selff)

    v1 = view(s[0:B], c[0:B])
    v2 = view(s[B:2 * B], c[B:2 * B])

    def proj(v, wa, ba, wb, bb):
        h = jnp.maximum(jnp.dot(v, wa, precision=hi,
                                preferred_element_type=f32) + ba, 0.0)
        return jnp.dot(h, wb, precision=hi, preferred_element_type=f32) + bb

    z1 = proj(v1, w1a_ref[...], b1a_ref[...], w1b_ref[...], b1b_ref[...])
    z2 = proj(v2, w2a_ref[...], b2a_ref[...], w2b_ref[...], b2b_ref[...])

    def normalize(z):
        nrm = jnp.sqrt(jnp.sum(z * z, axis=1, keepdims=True))
        return z / jnp.maximum(nrm, 1e-12)

    z1n = normalize(z1)
    z2n = normalize(z2)
    sim = lax.dot_general(z1n, z2n, (((1,), (1,)), ((), ())),
                          precision=hi, preferred_element_type=f32) / TEMP

    mx_r = jnp.max(sim, axis=1, keepdims=True)
    lse_r = mx_r[:, 0] + jnp.log(jnp.sum(jnp.exp(sim - mx_r), axis=1))
    mx_c = jnp.max(sim, axis=0, keepdims=True)
    lse_c = mx_c[0, :] + jnp.log(jnp.sum(jnp.exp(sim - mx_c), axis=0))
    eye = (lax.broadcasted_iota(jnp.int32, (B, B), 0) == pcols).astype(f32)
    diag_mean = jnp.sum(sim * eye) / B
    loss = 0.5 * (jnp.mean(lse_r) + jnp.mean(lse_c)) - diag_mean
    out_ref[...] = jnp.broadcast_to(loss, (1, 1))


def kernel(node_features, edge_index, edge_rel, batch_node_ids, rel_emb,
           W1a, b1a, W1b, b1b, W2a, b2a, W2b, b2b):
    nf = node_features.astype(jnp.float32)
    dst = jnp.pad(edge_index[0].astype(jnp.int32), (0, EPAD - E))
    nbr = jnp.pad(edge_index[1].astype(jnp.int32), (0, EPAD - E))
    rel = jnp.pad(edge_rel.astype(jnp.int32), (0, EPAD - E))
    bids = batch_node_ids.astype(jnp.int32)
    neg1 = jnp.full((N,), -1, jnp.int32)
    zrows = jnp.zeros((RPT, D), jnp.float32)
    z128 = jnp.zeros((RPT, 256), jnp.float32)

    sums, cnt, slotp, selff = _make_sc_aggregate()(nf, dst, nbr, rel, bids,
                                                   neg1, zrows, z128)

    rel16 = jnp.zeros((16, D), jnp.float32).at[:rel_emb.shape[0]].set(rel_emb)
    loss = pl.pallas_call(
        _tc_head,
        out_shape=jax.ShapeDtypeStruct((1, 1), jnp.float32),
    )(sums.reshape(NC, ROWS, D), cnt.reshape(NC, ROWS, 256),
      slotp.reshape(B, 1), selff, rel16,
      W1a, b1a.reshape(1, D), W1b, b1b.reshape(1, D // 2),
      W2a, b2a.reshape(1, D), W2b, b2b.reshape(1, D // 2))
    return loss[0, 0]


# speedup vs baseline: 14.4096x; 2.1494x over previous
import functools

import jax
import jax.numpy as jnp
from jax import lax
from jax.experimental import pallas as pl
from jax.experimental.pallas import tpu as pltpu
from jax.experimental.pallas import tpu_sc as plsc

NC = 2
NS = 16
L = 16
NW = NC * NS

N = 10000
E = 160000
D = 256
B = 1024
TEMP = 0.07

EPT = -(-E // NW)
EPT_PAD = -(-EPT // L) * L
EPAD = NW * EPT_PAD
NGROUPS = EPT_PAD // L

OROWS = 2 * B
SPT = OROWS // NS
LOC_JUNK = SPT
JUNK = OROWS * 16
K = 64
LIST_LEN = EPT_PAD + 112
PPT = B // NW


def _sc_body(nf, dst, nbr, rel, bids, neg1, zrows,
             out_sums, out_cnt, out_slot, out_self,
             slot_tab, bids_v, dst_v, nbr_v, rel_v, pkd_c,
             fin_pkd, hdr_v, hdr_all,
             rows_a, acc, cnt_acc, sf_rows, nbr_b,
             slotp_v, posi_v, sh_pkd, sh_hdr, sem1, sem2):
    sid = lax.axis_index("s")
    cid = lax.axis_index("c")
    wid = sid * NC + cid
    lane = lax.iota(jnp.int32, L)

    pltpu.sync_copy(zrows, acc)
    pltpu.sync_copy(neg1, slot_tab)
    pltpu.sync_copy(bids, bids_v)

    def zero_cnt(g, carry):
        cnt_acc[g, pl.ds(0, L)] = jnp.zeros((L,), jnp.float32)
        return carry

    lax.fori_loop(0, SPT + 8, zero_cnt, 0)

    def build(g, carry):
        bv = bids_v[pl.ds(g * L, L)]
        vals = g * L + lane
        plsc.store_scatter(slot_tab, [bv], vals)
        return carry

    lax.fori_loop(0, B // L, build, 0)

    ebase = wid * EPT_PAD
    pltpu.sync_copy(dst.at[pl.ds(ebase, EPT_PAD)], dst_v)
    pltpu.sync_copy(nbr.at[pl.ds(ebase, EPT_PAD)], nbr_v)
    pltpu.sync_copy(rel.at[pl.ds(ebase, EPT_PAD)], rel_v)

    def scan(j, off):
        base = j * L
        dstv = dst_v[pl.ds(base, L)]
        relv = rel_v[pl.ds(base, L)]
        valid = (ebase + base + lane) < E
        dcl = jnp.minimum(jnp.maximum(dstv, 0), N - 1)
        slots = plsc.load_gather(slot_tab, [dcl])
        hit = valid & (slots >= 0)
        v1m = (relv == 0) | (relv >= 7)
        tgt = slots + jnp.where(v1m, 0, B)
        nbrv = nbr_v[pl.ds(base, L)]
        pkd = (nbrv << 15) | (tgt << 4) | relv
        plsc.store_compressed(pkd_c.at[pl.ds(off, L)], pkd, mask=hit)
        nhit = plsc.all_reduce_population_count(hit)
        return off + nhit[0]

    off = lax.fori_loop(0, NGROUPS, scan, jnp.int32(0))

    pltpu.sync_copy(pkd_c, sh_pkd.at[pl.ds(sid * LIST_LEN, LIST_LEN)])
    hdr_v[pl.ds(0, L)] = jnp.broadcast_to(off, (L,)).astype(jnp.int32)
    pltpu.sync_copy(hdr_v, sh_hdr.at[pl.ds(sid * L, L)])
    plsc.subcore_barrier()
    pltpu.sync_copy(sh_hdr, hdr_all)

    def process_batch(start):
        for t in range(K // L):
            pk0 = fin_pkd[pl.ds(start + t * L, L)]
            nbr_b[pl.ds(t * L, L)] = jnp.where(pk0 < 0, 0, pk0 >> 15)
        pltpu.async_copy(nf.at[nbr_b], rows_a, sem1).wait()

        def accum_group(t, carry):
            pk = fin_pkd[pl.ds(start + t * L, L)]
            tgt = (pk >> 4) & (OROWS - 1)
            loc = jnp.where(pk < 0, LOC_JUNK, tgt & (SPT - 1))
            relv = pk & 15
            for l in range(L):
                li = loc[l]
                for col in range(D // L):
                    cs = pl.ds(col * L, L)
                    acc[li, cs] = acc[li, cs] + rows_a[t * L + l, cs]
                oh = (lane == relv[l]).astype(jnp.float32)
                cnt_acc[li, pl.ds(0, L)] = cnt_acc[li, pl.ds(0, L)] + oh
            return carry

        lax.fori_loop(0, K // L, accum_group, 0)

    def peer_body(p, off_f):
        pltpu.sync_copy(sh_pkd.at[pl.ds(p * LIST_LEN, LIST_LEN)], pkd_c)
        off_p = hdr_all[pl.ds(p * L, L)][0]
        ng = (off_p + L - 1) // L

        def scan_peer(j, ofs):
            pk = pkd_c[pl.ds(j * L, L)]
            ok = ((j * L + lane) < off_p) & (((pk >> 11) & 15) == sid)
            plsc.store_compressed(fin_pkd.at[pl.ds(ofs, L)], pk, mask=ok)
            nok = plsc.all_reduce_population_count(ok)
            return ofs + nok[0]

        off_f = lax.fori_loop(0, ng, scan_peer, off_f)

        def flush_body(b, carry):
            process_batch(b * K)
            return carry

        nfull = off_f // K
        lax.fori_loop(0, nfull, flush_body, 0)
        done = nfull * K
        for t in range(K // L):
            vp = fin_pkd[pl.ds(done + t * L, L)]
            fin_pkd[pl.ds(t * L, L)] = vp
        return off_f - done

    off_f = lax.fori_loop(0, NS, peer_body, jnp.int32(0))
    for t in range(K // L):
        fin_pkd[pl.ds(off_f + t * L, L)] = jnp.full((L,), -1, jnp.int32)
    process_batch(0)

    pltpu.sync_copy(acc.at[pl.ds(0, SPT)],
                    out_sums.at[pl.ds(cid * OROWS + sid * SPT, SPT)])
    pltpu.sync_copy(cnt_acc.at[pl.ds(0, SPT)],
                    out_cnt.at[pl.ds(cid * OROWS + sid * SPT, SPT)])

    pbase = wid * PPT
    for t in range(PPT // L):
        bv = bids_v[pl.ds(pbase + t * L, L)]
        slotp_v[pl.ds(t * L, L)] = plsc.load_gather(slot_tab, [bv])
        posi_v[pl.ds(t * L, L)] = bv
    pltpu.sync_copy(slotp_v, out_slot.at[pl.ds(pbase, PPT)])
    pltpu.async_copy(nf.at[posi_v], sf_rows, sem2).wait()
    pltpu.sync_copy(sf_rows, out_self.at[pl.ds(pbase, PPT)])


_sc_aggregate = None


def _make_sc_aggregate():
    global _sc_aggregate
    if _sc_aggregate is None:
        _sc_aggregate = pl.kernel(
            _sc_body,
            out_type=(
                jax.ShapeDtypeStruct((NC * OROWS, D), jnp.float32),
                jax.ShapeDtypeStruct((NC * OROWS, L), jnp.float32),
                jax.ShapeDtypeStruct((B,), jnp.int32),
                jax.ShapeDtypeStruct((B, D), jnp.float32),
            ),
            mesh=plsc.VectorSubcoreMesh(core_axis_name="c",
                                        subcore_axis_name="s",
                                        num_cores=NC, num_subcores=NS),
            scratch_types=(
                pltpu.VMEM((N,), jnp.int32),
                pltpu.VMEM((B,), jnp.int32),
                pltpu.VMEM((EPT_PAD,), jnp.int32),
                pltpu.VMEM((EPT_PAD,), jnp.int32),
                pltpu.VMEM((EPT_PAD,), jnp.int32),
                pltpu.VMEM((LIST_LEN,), jnp.int32),
                pltpu.VMEM((LIST_LEN,), jnp.int32),
                pltpu.VMEM((L,), jnp.int32),
                pltpu.VMEM((NS * L,), jnp.int32),
                pltpu.VMEM((K, D), jnp.float32),
                pltpu.VMEM((SPT + 8, D), jnp.float32),
                pltpu.VMEM((SPT + 8, L), jnp.float32),
                pltpu.VMEM((PPT, D), jnp.float32),
                pltpu.VMEM((K,), jnp.int32),
                pltpu.VMEM((PPT,), jnp.int32),
                pltpu.VMEM((PPT,), jnp.int32),
                pltpu.VMEM_SHARED((NS * LIST_LEN,), jnp.int32),
                pltpu.VMEM_SHARED((NS * L,), jnp.int32),
                pltpu.SemaphoreType.DMA,
                pltpu.SemaphoreType.DMA,
            ),
            compiler_params=pltpu.CompilerParams(needs_layout_passes=False),
        )
    return _sc_aggregate


def _tc_head(sums_ref, cnt_ref, slotp_ref, self_ref, rel16_ref,
             w1a_ref, b1a_ref, w1b_ref, b1b_ref,
             w2a_ref, b2a_ref, w2b_ref, b2b_ref, out_ref):
    f32 = jnp.float32
    hi = lax.Precision.HIGHEST

    s = sums_ref[0] + sums_ref[1]
    c = cnt_ref[0] + cnt_ref[1]
    slotp = slotp_ref[...]
    selff = self_ref[...]
    rel16 = rel16_ref[...]

    pcols = lax.broadcasted_iota(jnp.int32, (B, B), 1)
    P = (slotp == pcols).astype(f32)

    def view(s_rows, c_rows):
        sp = jnp.dot(P, s_rows, precision=hi, preferred_element_type=f32)
        cp = jnp.dot(P, c_rows, precision=hi, preferred_element_type=f32)
        r = jnp.dot(cp, rel16, precision=hi,
                    preferred_element_type=f32)
        n = jnp.sum(cp, axis=1, keepdims=True)
        return jnp.where(n > 0.0, (sp + r) / jnp.maximum(n, 1.0), selff)

    v1 = view(s[0:B], c[0:B])
    v2 = view(s[B:2 * B], c[B:2 * B])

    def proj(v, wa, ba, wb, bb):
        h = jnp.maximum(jnp.dot(v, wa, precision=hi,
                                preferred_element_type=f32) + ba, 0.0)
        return jnp.dot(h, wb, precision=hi, preferred_element_type=f32) + bb

    z1 = proj(v1, w1a_ref[...], b1a_ref[...], w1b_ref[...], b1b_ref[...])
    z2 = proj(v2, w2a_ref[...], b2a_ref[...], w2b_ref[...], b2b_ref[...])

    def normalize(z):
        nrm = jnp.sqrt(jnp.sum(z * z, axis=1, keepdims=True))
        return z / jnp.maximum(nrm, 1e-12)

    z1n = normalize(z1)
    z2n = normalize(z2)
    sim = lax.dot_general(z1n, z2n, (((1,), (1,)), ((), ())),
                          precision=hi, preferred_element_type=f32) / TEMP

    mx_r = jnp.max(sim, axis=1, keepdims=True)
    lse_r = mx_r[:, 0] + jnp.log(jnp.sum(jnp.exp(sim - mx_r), axis=1))
    mx_c = jnp.max(sim, axis=0, keepdims=True)
    lse_c = mx_c[0, :] + jnp.log(jnp.sum(jnp.exp(sim - mx_c), axis=0))
    eye = (lax.broadcasted_iota(jnp.int32, (B, B), 0) == pcols).astype(f32)
    diag_mean = jnp.sum(sim * eye) / B
    loss = 0.5 * (jnp.mean(lse_r) + jnp.mean(lse_c)) - diag_mean
    out_ref[...] = jnp.broadcast_to(loss, (1, 1))


def kernel(node_features, edge_index, edge_rel, batch_node_ids, rel_emb,
           W1a, b1a, W1b, b1b, W2a, b2a, W2b, b2b):
    nf = node_features.astype(jnp.float32)
    dst = jnp.pad(edge_index[0].astype(jnp.int32), (0, EPAD - E))
    nbr = jnp.pad(edge_index[1].astype(jnp.int32), (0, EPAD - E))
    rel = jnp.pad(edge_rel.astype(jnp.int32), (0, EPAD - E))
    bids = batch_node_ids.astype(jnp.int32)
    neg1 = jnp.full((N,), -1, jnp.int32)
    zrows = jnp.zeros((SPT + 8, D), jnp.float32)

    sums, cnt, slotp, selff = _make_sc_aggregate()(nf, dst, nbr, rel, bids,
                                                   neg1, zrows)

    rel16 = jnp.zeros((16, D), jnp.float32).at[:rel_emb.shape[0]].set(rel_emb)
    loss = pl.pallas_call(
        _tc_head,
        out_shape=jax.ShapeDtypeStruct((1, 1), jnp.float32),
    )(sums.reshape(NC, OROWS, D), cnt.reshape(NC, OROWS, L),
      slotp.reshape(B, 1), selff, rel16,
      W1a, b1a.reshape(1, D), W1b, b1b.reshape(1, D // 2),
      W2a, b2a.reshape(1, D), W2b, b2b.reshape(1, D // 2))
    return loss[0, 0]
